# Initial kernel scaffold; baseline (speedup 1.0000x reference)
import jax
import jax.numpy as jnp
from jax.experimental import pallas as pl


def kernel(x, edge_index, W1l, b1l, W1r, W2l, b2l, W2r):
    raise NotImplementedError("write your pallas kernel here")


# speedup vs baseline: 3.0990x; 3.0990x over previous
import functools

import jax
import jax.numpy as jnp
from jax import lax
from jax.experimental import pallas as pl
from jax.experimental.pallas import tpu as pltpu
from jax.experimental.pallas import tpu_sc as plsc

N = 10000
E = 320000
D = 128

NC = 2
NS = 16
L = 16
NW = NC * NS
K = 128
R = 4
_C0 = (E + NW * K - 1) // (NW * K)
C = ((_C0 + R - 1) // R) * R
CR = C // R
EPAD = NW * C * K
NPAD = 10240
STRIPE = NPAD // NS

BLK = 1000
GRID = N // BLK


def _tc_pre_body(x_ref, wl_ref, wr_ref, b_ref, z_ref, r_ref):
    xb = x_ref[...]
    z_ref[...] = jnp.dot(xb, wl_ref[...], preferred_element_type=jnp.float32)
    r_ref[...] = (jnp.dot(xb, wr_ref[...], preferred_element_type=jnp.float32)
                  + b_ref[...])


def _tc_pre(x, wl, wr, b):
    return pl.pallas_call(
        _tc_pre_body,
        grid=(GRID,),
        in_specs=[
            pl.BlockSpec((BLK, D), lambda i: (i, 0)),
            pl.BlockSpec((D, D), lambda i: (0, 0)),
            pl.BlockSpec((D, D), lambda i: (0, 0)),
            pl.BlockSpec((1, D), lambda i: (0, 0)),
        ],
        out_specs=[
            pl.BlockSpec((BLK, D), lambda i: (i, 0)),
            pl.BlockSpec((BLK, D), lambda i: (i, 0)),
        ],
        out_shape=[
            jax.ShapeDtypeStruct((N, D), jnp.float32),
            jax.ShapeDtypeStruct((N, D), jnp.float32),
        ],
    )(x, wl, wr, b)


def _tc_mid_body(agg_ref, degp_ref, r1_ref, wl_ref, wr_ref, b_ref,
                 z2_ref, r2_ref):
    deg = (degp_ref[0] + degp_ref[1])[:, 0:1]
    rdeg = 1.0 / jnp.maximum(deg, 1.0)
    mean = (agg_ref[0] + agg_ref[1]) * rdeg
    h = jnp.maximum(mean + r1_ref[...], 0.0)
    z2_ref[...] = jnp.dot(h, wl_ref[...], preferred_element_type=jnp.float32)
    r2_ref[...] = (jnp.dot(h, wr_ref[...], preferred_element_type=jnp.float32)
                   + b_ref[...])


def _tc_mid(agg, degp, r1, wl, wr, b):
    return pl.pallas_call(
        _tc_mid_body,
        grid=(GRID,),
        in_specs=[
            pl.BlockSpec((NC, BLK, D), lambda i: (0, i, 0)),
            pl.BlockSpec((NC, BLK, D), lambda i: (0, i, 0)),
            pl.BlockSpec((BLK, D), lambda i: (i, 0)),
            pl.BlockSpec((D, D), lambda i: (0, 0)),
            pl.BlockSpec((D, D), lambda i: (0, 0)),
            pl.BlockSpec((1, D), lambda i: (0, 0)),
        ],
        out_specs=[
            pl.BlockSpec((BLK, D), lambda i: (i, 0)),
            pl.BlockSpec((BLK, D), lambda i: (i, 0)),
        ],
        out_shape=[
            jax.ShapeDtypeStruct((N, D), jnp.float32),
            jax.ShapeDtypeStruct((N, D), jnp.float32),
        ],
    )(agg, degp, r1, wl, wr, b)


def _tc_post_body(agg_ref, degp_ref, r2_ref, out_ref):
    deg = (degp_ref[0] + degp_ref[1])[:, 0:1]
    rdeg = 1.0 / jnp.maximum(deg, 1.0)
    o = (agg_ref[0] + agg_ref[1]) * rdeg + r2_ref[...]
    m = jnp.max(o, axis=-1, keepdims=True)
    lse = jnp.log(jnp.sum(jnp.exp(o - m), axis=-1, keepdims=True)) + m
    out_ref[...] = o - lse


def _tc_post(agg, degp, r2):
    return pl.pallas_call(
        _tc_post_body,
        grid=(GRID,),
        in_specs=[
            pl.BlockSpec((NC, BLK, D), lambda i: (0, i, 0)),
            pl.BlockSpec((NC, BLK, D), lambda i: (0, i, 0)),
            pl.BlockSpec((BLK, D), lambda i: (i, 0)),
        ],
        out_specs=pl.BlockSpec((BLK, D), lambda i: (i, 0)),
        out_shape=jax.ShapeDtypeStruct((N, D), jnp.float32),
    )(agg, degp, r2)


def _sc_agg(z, srcw, dstw, zrow):
    mesh = plsc.VectorSubcoreMesh(core_axis_name="c", subcore_axis_name="s")

    def body(z_hbm, srcw_hbm, dstw_hbm, zrow_hbm, agg_out,
             src_all, dst_all, rows, sem, agg_sp):
        cid = lax.axis_index("c")
        sid = lax.axis_index("s")
        wid = cid * NS + sid

        pltpu.sync_copy(zrow_hbm, agg_sp.at[pl.ds(sid * STRIPE, STRIPE)])
        plsc.subcore_barrier()

        def rnd(r, carry):
            pltpu.sync_copy(srcw_hbm.at[wid].at[r], src_all)
            pltpu.sync_copy(dstw_hbm.at[wid].at[r], dst_all)

            def chunk(j, c2):
                pltpu.async_copy(z_hbm.at[src_all.at[j]], rows, sem).wait()
                pltpu.sync_copy(rows, agg_sp.at[dst_all.at[j]], add=True)
                return c2
            return lax.fori_loop(0, CR, chunk, carry)
        lax.fori_loop(0, R, rnd, 0)

        plsc.subcore_barrier()
        pltpu.sync_copy(agg_sp.at[pl.ds(sid * STRIPE, STRIPE)],
                        agg_out.at[cid].at[pl.ds(sid * STRIPE, STRIPE)])

    run = pl.kernel(
        body,
        out_type=jax.ShapeDtypeStruct((NC, NPAD, D), jnp.float32),
        mesh=mesh,
        scratch_types=(
            pltpu.VMEM((CR, K), jnp.int32),
            pltpu.VMEM((CR, K), jnp.int32),
            pltpu.VMEM((K, D), jnp.float32),
            pltpu.SemaphoreType.DMA,
            pltpu.VMEM_SHARED((NPAD, D), jnp.float32),
        ),
    )
    return run(z, srcw, dstw, zrow)


def _sc_deg(dstw, zrow, ones_in):
    mesh = plsc.VectorSubcoreMesh(core_axis_name="c", subcore_axis_name="s")

    def body(dstw_hbm, zrow_hbm, ones_hbm, deg_out,
             dst_all, ones_v, deg_sp):
        cid = lax.axis_index("c")
        sid = lax.axis_index("s")
        wid = cid * NS + sid

        pltpu.sync_copy(zrow_hbm, deg_sp.at[pl.ds(sid * STRIPE, STRIPE)])
        pltpu.sync_copy(ones_hbm, ones_v)
        plsc.subcore_barrier()

        def rnd(r, carry):
            pltpu.sync_copy(dstw_hbm.at[wid].at[r], dst_all)

            def chunk(j, c2):
                pltpu.sync_copy(ones_v, deg_sp.at[dst_all.at[j]], add=True)
                return c2
            return lax.fori_loop(0, CR, chunk, carry)
        lax.fori_loop(0, R, rnd, 0)

        plsc.subcore_barrier()
        pltpu.sync_copy(deg_sp.at[pl.ds(sid * STRIPE, STRIPE)],
                        deg_out.at[cid].at[pl.ds(sid * STRIPE, STRIPE)])

    run = pl.kernel(
        body,
        out_type=jax.ShapeDtypeStruct((NC, NPAD, D), jnp.float32),
        mesh=mesh,
        scratch_types=(
            pltpu.VMEM((CR, K), jnp.int32),
            pltpu.VMEM((K, D), jnp.float32),
            pltpu.VMEM_SHARED((NPAD, D), jnp.float32),
        ),
    )
    return run(dstw, zrow, ones_in)


def kernel(x, edge_index, W1l, b1l, W1r, W2l, b2l, W2r):
    src = edge_index[0].astype(jnp.int32)
    dst = edge_index[1].astype(jnp.int32)
    src = jnp.concatenate([src, jnp.zeros((EPAD - E,), jnp.int32)])
    dst = jnp.concatenate([dst, jnp.full((EPAD - E,), N, jnp.int32)])
    srcw = src.reshape(NW, R, CR, K)
    dstw = dst.reshape(NW, R, CR, K)
    zrow = jnp.zeros((STRIPE, D), jnp.float32)
    ones_in = jnp.ones((K, D), jnp.float32)

    b1 = b1l.reshape(1, D)
    b2 = b2l.reshape(1, D)

    degp = _sc_deg(dstw, zrow, ones_in)
    z1, r1 = _tc_pre(x, W1l, W1r, b1)
    agg1 = _sc_agg(z1, srcw, dstw, zrow)
    z2, r2 = _tc_mid(agg1, degp, r1, W2l, W2r, b2)
    agg2 = _sc_agg(z2, srcw, dstw, zrow)
    return _tc_post(agg2, degp, r2)


# speedup vs baseline: 3.4340x; 1.1081x over previous
import functools

import jax
import jax.numpy as jnp
from jax import lax
from jax.experimental import pallas as pl
from jax.experimental.pallas import tpu as pltpu
from jax.experimental.pallas import tpu_sc as plsc

N = 10000
E = 320000
D = 128

NC = 2
NS = 16
L = 16
NW = NC * NS
K = 128
R = 4
_C0 = (E + NW * K - 1) // (NW * K)
C = ((_C0 + R - 1) // R) * R
CR = C // R
EPAD = NW * C * K
NPAD = 10240
STRIPE = NPAD // NS

BLK = 1000
GRID = N // BLK


def _tc_pre_body(x_ref, wl_ref, wr_ref, b_ref, z_ref, r_ref):
    xb = x_ref[...]
    z_ref[...] = jnp.dot(xb, wl_ref[...], preferred_element_type=jnp.float32)
    r_ref[...] = (jnp.dot(xb, wr_ref[...], preferred_element_type=jnp.float32)
                  + b_ref[...])


def _tc_pre(x, wl, wr, b):
    return pl.pallas_call(
        _tc_pre_body,
        grid=(GRID,),
        in_specs=[
            pl.BlockSpec((BLK, D), lambda i: (i, 0)),
            pl.BlockSpec((D, D), lambda i: (0, 0)),
            pl.BlockSpec((D, D), lambda i: (0, 0)),
            pl.BlockSpec((1, D), lambda i: (0, 0)),
        ],
        out_specs=[
            pl.BlockSpec((BLK, D), lambda i: (i, 0)),
            pl.BlockSpec((BLK, D), lambda i: (i, 0)),
        ],
        out_shape=[
            jax.ShapeDtypeStruct((N, D), jnp.float32),
            jax.ShapeDtypeStruct((N, D), jnp.float32),
        ],
    )(x, wl, wr, b)


def _tc_mid_body(agg_ref, degp_ref, r1_ref, wl_ref, wr_ref, b_ref,
                 z2_ref, r2_ref):
    deg = (degp_ref[0] + degp_ref[1])[:, 0:1]
    rdeg = 1.0 / jnp.maximum(deg, 1.0)
    mean = (agg_ref[0] + agg_ref[1]) * rdeg
    h = jnp.maximum(mean + r1_ref[...], 0.0)
    z2_ref[...] = jnp.dot(h, wl_ref[...], preferred_element_type=jnp.float32)
    r2_ref[...] = (jnp.dot(h, wr_ref[...], preferred_element_type=jnp.float32)
                   + b_ref[...])


def _tc_mid(agg, degp, r1, wl, wr, b):
    return pl.pallas_call(
        _tc_mid_body,
        grid=(GRID,),
        in_specs=[
            pl.BlockSpec((NC, BLK, D), lambda i: (0, i, 0)),
            pl.BlockSpec((NC, BLK, D), lambda i: (0, i, 0)),
            pl.BlockSpec((BLK, D), lambda i: (i, 0)),
            pl.BlockSpec((D, D), lambda i: (0, 0)),
            pl.BlockSpec((D, D), lambda i: (0, 0)),
            pl.BlockSpec((1, D), lambda i: (0, 0)),
        ],
        out_specs=[
            pl.BlockSpec((BLK, D), lambda i: (i, 0)),
            pl.BlockSpec((BLK, D), lambda i: (i, 0)),
        ],
        out_shape=[
            jax.ShapeDtypeStruct((N, D), jnp.float32),
            jax.ShapeDtypeStruct((N, D), jnp.float32),
        ],
    )(agg, degp, r1, wl, wr, b)


def _tc_post_body(agg_ref, degp_ref, r2_ref, out_ref):
    deg = (degp_ref[0] + degp_ref[1])[:, 0:1]
    rdeg = 1.0 / jnp.maximum(deg, 1.0)
    o = (agg_ref[0] + agg_ref[1]) * rdeg + r2_ref[...]
    m = jnp.max(o, axis=-1, keepdims=True)
    lse = jnp.log(jnp.sum(jnp.exp(o - m), axis=-1, keepdims=True)) + m
    out_ref[...] = o - lse


def _tc_post(agg, degp, r2):
    return pl.pallas_call(
        _tc_post_body,
        grid=(GRID,),
        in_specs=[
            pl.BlockSpec((NC, BLK, D), lambda i: (0, i, 0)),
            pl.BlockSpec((NC, BLK, D), lambda i: (0, i, 0)),
            pl.BlockSpec((BLK, D), lambda i: (i, 0)),
        ],
        out_specs=pl.BlockSpec((BLK, D), lambda i: (i, 0)),
        out_shape=jax.ShapeDtypeStruct((N, D), jnp.float32),
    )(agg, degp, r2)


def _sc_agg(z, srcw, dstw, zrow):
    mesh = plsc.VectorSubcoreMesh(core_axis_name="c", subcore_axis_name="s")

    def body(z_hbm, srcw_hbm, dstw_hbm, zrow_hbm, agg_out,
             src_all, dst_all, rows0, rows1, sem0, sem1, agg_sp):
        cid = lax.axis_index("c")
        sid = lax.axis_index("s")
        wid = cid * NS + sid

        pltpu.sync_copy(zrow_hbm, agg_sp.at[pl.ds(sid * STRIPE, STRIPE)])
        plsc.subcore_barrier()

        def rnd(r, carry):
            pltpu.sync_copy(srcw_hbm.at[wid].at[r], src_all)
            pltpu.sync_copy(dstw_hbm.at[wid].at[r], dst_all)
            pltpu.async_copy(z_hbm.at[src_all.at[0]], rows0, sem0)

            def pair(jj, c2):
                j0 = jj * 2
                pltpu.async_copy(z_hbm.at[src_all.at[j0 + 1]], rows1, sem1)
                pltpu.make_async_copy(z_hbm.at[src_all.at[j0]],
                                      rows0, sem0).wait()
                pltpu.sync_copy(rows0, agg_sp.at[dst_all.at[j0]], add=True)

                @pl.when(jj < CR // 2 - 1)
                def _():
                    pltpu.async_copy(z_hbm.at[src_all.at[j0 + 2]],
                                     rows0, sem0)
                pltpu.make_async_copy(z_hbm.at[src_all.at[j0 + 1]],
                                      rows1, sem1).wait()
                pltpu.sync_copy(rows1, agg_sp.at[dst_all.at[j0 + 1]],
                                add=True)
                return c2
            return lax.fori_loop(0, CR // 2, pair, carry)
        lax.fori_loop(0, R, rnd, 0)

        plsc.subcore_barrier()
        pltpu.sync_copy(agg_sp.at[pl.ds(sid * STRIPE, STRIPE)],
                        agg_out.at[cid].at[pl.ds(sid * STRIPE, STRIPE)])

    run = pl.kernel(
        body,
        out_type=jax.ShapeDtypeStruct((NC, NPAD, D), jnp.float32),
        mesh=mesh,
        scratch_types=(
            pltpu.VMEM((CR, K), jnp.int32),
            pltpu.VMEM((CR, K), jnp.int32),
            pltpu.VMEM((K, D), jnp.float32),
            pltpu.VMEM((K, D), jnp.float32),
            pltpu.SemaphoreType.DMA,
            pltpu.SemaphoreType.DMA,
            pltpu.VMEM_SHARED((NPAD, D), jnp.float32),
        ),
    )
    return run(z, srcw, dstw, zrow)


def _sc_deg(dstw, zrow, ones_in):
    mesh = plsc.VectorSubcoreMesh(core_axis_name="c", subcore_axis_name="s")

    def body(dstw_hbm, zrow_hbm, ones_hbm, deg_out,
             dst_all, ones_v, deg_sp):
        cid = lax.axis_index("c")
        sid = lax.axis_index("s")
        wid = cid * NS + sid

        pltpu.sync_copy(zrow_hbm, deg_sp.at[pl.ds(sid * STRIPE, STRIPE)])
        pltpu.sync_copy(ones_hbm, ones_v)
        plsc.subcore_barrier()

        def rnd(r, carry):
            pltpu.sync_copy(dstw_hbm.at[wid].at[r], dst_all)

            def chunk(j, c2):
                pltpu.sync_copy(ones_v, deg_sp.at[dst_all.at[j]], add=True)
                return c2
            return lax.fori_loop(0, CR, chunk, carry)
        lax.fori_loop(0, R, rnd, 0)

        plsc.subcore_barrier()
        pltpu.sync_copy(deg_sp.at[pl.ds(sid * STRIPE, STRIPE)],
                        deg_out.at[cid].at[pl.ds(sid * STRIPE, STRIPE)])

    run = pl.kernel(
        body,
        out_type=jax.ShapeDtypeStruct((NC, NPAD, D), jnp.float32),
        mesh=mesh,
        scratch_types=(
            pltpu.VMEM((CR, K), jnp.int32),
            pltpu.VMEM((K, D), jnp.float32),
            pltpu.VMEM_SHARED((NPAD, D), jnp.float32),
        ),
    )
    return run(dstw, zrow, ones_in)


def kernel(x, edge_index, W1l, b1l, W1r, W2l, b2l, W2r):
    src = edge_index[0].astype(jnp.int32)
    dst = edge_index[1].astype(jnp.int32)
    src = jnp.concatenate([src, jnp.zeros((EPAD - E,), jnp.int32)])
    dst = jnp.concatenate([dst, jnp.full((EPAD - E,), N, jnp.int32)])
    srcw = src.reshape(NW, R, CR, K)
    dstw = dst.reshape(NW, R, CR, K)
    zrow = jnp.zeros((STRIPE, D), jnp.float32)
    ones_in = jnp.ones((K, D), jnp.float32)

    b1 = b1l.reshape(1, D)
    b2 = b2l.reshape(1, D)

    degp = _sc_deg(dstw, zrow, ones_in)
    z1, r1 = _tc_pre(x, W1l, W1r, b1)
    agg1 = _sc_agg(z1, srcw, dstw, zrow)
    z2, r2 = _tc_mid(agg1, degp, r1, W2l, W2r, b2)
    agg2 = _sc_agg(z2, srcw, dstw, zrow)
    return _tc_post(agg2, degp, r2)
